# baseline (speedup 1.0000x reference)
import functools

import jax
import jax.numpy as jnp
from jax import lax
from jax.experimental import pallas as pl
from jax.experimental.pallas import tpu as pltpu
from jax.experimental.pallas import tpu_sc as plsc

_B, _S, _D = 4, 4096, 2048
_NC, _NS = 2, 16
_NW = _NC * _NS
_DCHUNKS = _NW // _B
_DW = _D // _DCHUNKS
_L = 16
_JV = _DW // _L
_R = 128
_NCHUNK = _S // _R

_mesh = plsc.VectorSubcoreMesh(core_axis_name="c", subcore_axis_name="s")


@functools.partial(
    pl.kernel,
    out_type=jax.ShapeDtypeStruct((_B, _S, _D), jnp.float32),
    mesh=_mesh,
    scratch_types=[
        pltpu.VMEM((_R, _DW), jnp.float32),
        pltpu.VMEM((_R, _DW), jnp.float32),
        pltpu.SemaphoreType.DMA,
        pltpu.SemaphoreType.DMA,
    ],
)
def _sc_cumsum(x_hbm, o_hbm, buf0, buf1, sem0, sem1):
    wid = lax.axis_index("s") * _NC + lax.axis_index("c")
    b = wid // _DCHUNKS
    d0 = (wid % _DCHUNKS) * _DW
    bufs = (buf0, buf1)
    sems = (sem0, sem1)

    def in_slice(g):
        return x_hbm.at[b, pl.ds(g * _R, _R), pl.ds(d0, _DW)]

    pltpu.make_async_copy(in_slice(0), bufs[0], sems[0]).start()

    carry = tuple(jnp.zeros((_L,), jnp.float32) for _ in range(_JV))
    for g in range(_NCHUNK):
        buf = bufs[g % 2]
        if g + 1 < _NCHUNK:
            pltpu.make_async_copy(
                in_slice(g + 1), bufs[(g + 1) % 2], sems[(g + 1) % 2]
            ).start()
        pltpu.make_async_copy(in_slice(g), buf, sems[g % 2]).wait()

        def row_body(r, c):
            new = []
            for j in range(_JV):
                cj = c[j] + buf[r, pl.ds(j * _L, _L)]
                buf[r, pl.ds(j * _L, _L)] = cj
                new.append(cj)
            return tuple(new)

        carry = lax.fori_loop(0, _R, row_body, carry)
        pltpu.sync_copy(buf, o_hbm.at[b, pl.ds(g * _R, _R), pl.ds(d0, _DW)])


def kernel(input):
    return _sc_cumsum(input)
